# baseline (speedup 1.0000x reference)
import functools

import jax
import jax.numpy as jnp
from jax import lax
from jax.experimental import pallas as pl
from jax.experimental.pallas import tpu as pltpu
from jax.experimental.pallas import tpu_sc as plsc

_NUM_CORES = 2
_NUM_SUBCORES = 16
_NUM_WORKERS = _NUM_CORES * _NUM_SUBCORES
_LANES = 16

_BATCH = 4096
_SEQ = 200
_DIM = 64
_CHUNK = 100
_CHUNKS_PER_BATCH = _SEQ // _CHUNK
_B_PER_W = _BATCH // _NUM_WORKERS
_H_PER_W = _B_PER_W * _CHUNKS_PER_BATCH
_NBUF = 4
_NVEC = _DIM // _LANES


def _make_sc_call():
    mesh = plsc.VectorSubcoreMesh(core_axis_name="c", subcore_axis_name="s")

    @functools.partial(
        pl.kernel,
        mesh=mesh,
        compiler_params=pltpu.CompilerParams(use_tc_tiling_on_sc=False),
        out_type=jax.ShapeDtypeStruct((_BATCH, _DIM), jnp.float32),
        scratch_types=[
            pltpu.VMEM((_H_PER_W, _CHUNK), jnp.int32),
            pltpu.VMEM((_NBUF, _CHUNK, _DIM), jnp.float32),
            pltpu.VMEM((_B_PER_W, _DIM), jnp.float32),
            pltpu.SemaphoreType.DMA,
            pltpu.SemaphoreType.DMA,
            pltpu.SemaphoreType.DMA,
            pltpu.SemaphoreType.DMA,
        ],
    )
    def enc(tokens_hbm, table_hbm, out_hbm, idx_v, rows_v, out_v,
            sem0, sem1, sem2, sem3):
        sems = (sem0, sem1, sem2, sem3)
        wid = lax.axis_index("s") * _NUM_CORES + lax.axis_index("c")
        base_h = wid * _H_PER_W
        base_b = wid * _B_PER_W

        pltpu.make_async_copy(
            tokens_hbm.at[pl.ds(base_h, _H_PER_W)], idx_v, sem0).start()
        pltpu.make_async_copy(
            tokens_hbm.at[pl.ds(base_h, _H_PER_W)], idx_v, sem0).wait()

        def gather(h, buf):
            return pltpu.make_async_copy(
                table_hbm.at[idx_v.at[h]], rows_v.at[buf], sems[buf])

        for b in range(_NBUF):
            gather(jnp.int32(b), b).start()

        def reduce_chunk(buf, acc):
            rows = rows_v.at[buf]

            def body(i, carry):
                r = i * 5
                out = list(carry)
                for rr in range(5):
                    for k in range(_NVEC):
                        out[k] = out[k] + rows[r + rr, pl.ds(k * _LANES, _LANES)]
                return tuple(out)

            return lax.fori_loop(0, _CHUNK // 5, body, acc)

        inv_n = jnp.float32(1.0 / _SEQ)
        zeros = tuple(jnp.zeros((_LANES,), jnp.float32) for _ in range(_NVEC))

        def outer(i, carry):
            del carry
            for pair in range(_NBUF // _CHUNKS_PER_BATCH):
                acc = zeros
                for cb in range(_CHUNKS_PER_BATCH):
                    buf = pair * _CHUNKS_PER_BATCH + cb
                    h = i * _NBUF + buf
                    gather(h, buf).wait()
                    acc = reduce_chunk(buf, acc)
                    nxt = h + _NBUF

                    @pl.when(nxt < _H_PER_W)
                    def _():
                        gather(nxt, buf).start()

                b_local = i * (_NBUF // _CHUNKS_PER_BATCH) + pair
                for k in range(_NVEC):
                    out_v[b_local, pl.ds(k * _LANES, _LANES)] = acc[k] * inv_n
            return 0

        lax.fori_loop(0, _H_PER_W // _NBUF, outer, 0)

        pltpu.make_async_copy(
            out_v, out_hbm.at[pl.ds(base_b, _B_PER_W)], sem0).start()
        pltpu.make_async_copy(
            out_v, out_hbm.at[pl.ds(base_b, _B_PER_W)], sem0).wait()

    return enc


_sc_call = _make_sc_call()


def kernel(tokens, embedding_weight):
    tokens2 = tokens.reshape(_BATCH * _CHUNKS_PER_BATCH, _CHUNK)
    return _sc_call(tokens2, embedding_weight)
